# baseline (speedup 1.0000x reference)
import jax
import jax.numpy as jnp
from jax import lax
from jax.experimental import pallas as pl
from jax.experimental.pallas import tpu as pltpu
from jax.experimental.pallas import tpu_sc as plsc

N = 10000
E = 320000
E2 = E // 2
D_NODE = 128
D_EDGE = 16
H = 128
DEPTH = 4
N_GRAPHS = 64

S = 2
ES = E // S
E2S = E2 // S

SC_CORES = 2
SC_TILES = 16
NW = SC_CORES * SC_TILES
CHUNK = 40
W_E = ES // NW
W_CH = W_E // CHUNK
N_PAD = 10240
NROWS_T = N_PAD // SC_TILES

BE = 2000
BN = 2000


def _node_mm_body(x_ref, w_ref, o_ref):
    o_ref[...] = jnp.dot(x_ref[...], w_ref[...].astype(jnp.bfloat16),
                         preferred_element_type=jnp.float32)


def _node_mm(x_pad, w):
    return pl.pallas_call(
        _node_mm_body,
        grid=(N_PAD // 2048,),
        in_specs=[
            pl.BlockSpec((2048, D_NODE), lambda i: (i, 0)),
            pl.BlockSpec((D_NODE, H), lambda i: (0, 0)),
        ],
        out_specs=pl.BlockSpec((2048, H), lambda i: (i, 0)),
        out_shape=jax.ShapeDtypeStruct((N_PAD, H), jnp.float32),
    )(x_pad, w)


def _edge_init_body(xg_ref, ea_ref, wee_ref, be_ref, wc0_ref, h0_ref, hw_ref):
    h0 = jnp.maximum(
        xg_ref[0].astype(jnp.float32)
        + jnp.dot(ea_ref[0], wee_ref[...].astype(jnp.bfloat16),
                  preferred_element_type=jnp.float32)
        + be_ref[...],
        0.0,
    ).astype(jnp.bfloat16)
    h0_ref[0] = h0
    hw_ref[0] = jnp.dot(h0, wc0_ref[...].astype(jnp.bfloat16),
                        preferred_element_type=jnp.float32)


def _edge_init(xg, ea, w_ei_e, b_ei, w_c0):
    return pl.pallas_call(
        _edge_init_body,
        grid=(2, E2S // BE),
        in_specs=[
            pl.BlockSpec((1, BE, H), lambda g, e: (g, e, 0)),
            pl.BlockSpec((1, BE, D_EDGE), lambda g, e: (g, e, 0)),
            pl.BlockSpec((D_EDGE, H), lambda g, e: (0, 0)),
            pl.BlockSpec((1, H), lambda g, e: (0, 0)),
            pl.BlockSpec((H, H), lambda g, e: (0, 0)),
        ],
        out_specs=[
            pl.BlockSpec((1, BE, H), lambda g, e: (g, e, 0)),
            pl.BlockSpec((1, BE, H), lambda g, e: (g, e, 0)),
        ],
        out_shape=[
            jax.ShapeDtypeStruct((2, E2S, H), jnp.bfloat16),
            jax.ShapeDtypeStruct((2, E2S, H), jnp.float32),
        ],
    )(xg, ea, w_ei_e, b_ei, w_c0)


def _layer_body(g_ref, hwr_ref, h0_ref, b_ref, w_ref, o_ref):
    h = jnp.maximum(
        g_ref[0] - hwr_ref[0]
        + b_ref[...] + h0_ref[0].astype(jnp.float32),
        0.0,
    ).astype(jnp.bfloat16)
    o_ref[0] = jnp.dot(h, w_ref[...].astype(jnp.bfloat16),
                       preferred_element_type=jnp.float32)


def _layer(g, hw, h0, b, w_next):
    return pl.pallas_call(
        _layer_body,
        grid=(2, E2S // BE),
        in_specs=[
            pl.BlockSpec((1, BE, H), lambda g_, e: (g_, e, 0)),
            pl.BlockSpec((1, BE, H), lambda g_, e: (1 - g_, e, 0)),
            pl.BlockSpec((1, BE, H), lambda g_, e: (g_, e, 0)),
            pl.BlockSpec((1, H), lambda g_, e: (0, 0)),
            pl.BlockSpec((H, H), lambda g_, e: (0, 0)),
        ],
        out_specs=pl.BlockSpec((1, BE, H), lambda g_, e: (g_, e, 0)),
        out_shape=jax.ShapeDtypeStruct((2, E2S, H), jnp.float32),
    )(g, hw, h0, b, w_next)


def _combine4_body(a0_ref, a1_ref, b0_ref, b1_ref, o_ref):
    o_ref[...] = a0_ref[0] + a1_ref[0] + b0_ref[0] + b1_ref[0]


def _combine4(pa, pb):
    return pl.pallas_call(
        _combine4_body,
        grid=(N_PAD // 2048,),
        in_specs=[
            pl.BlockSpec((1, 2048, H), lambda i: (0, i, 0)),
            pl.BlockSpec((1, 2048, H), lambda i: (1, i, 0)),
            pl.BlockSpec((1, 2048, H), lambda i: (0, i, 0)),
            pl.BlockSpec((1, 2048, H), lambda i: (1, i, 0)),
        ],
        out_specs=pl.BlockSpec((2048, H), lambda i: (i, 0)),
        out_shape=jax.ShapeDtypeStruct((N_PAD, H), jnp.float32),
    )(pa, pa, pb, pb)


def _head_body(x_ref, tab_ref, batch_ref, wen_ref, ben_ref, w1_ref,
               b1_ref, w2_ref, b2_ref, o_ref, sums_ref, cnts_ref):
    i = pl.program_id(0)
    nb = pl.num_programs(0)
    hn = jnp.maximum(
        jnp.dot(x_ref[...], wen_ref[...], preferred_element_type=jnp.float32)
        + tab_ref[...] + ben_ref[...],
        0.0,
    )
    b = batch_ref[0, 0]
    p = (lax.broadcasted_iota(jnp.int32, (N_GRAPHS, BN), 0) == b[None, :]).astype(jnp.float32)
    psum = jnp.dot(p, hn, preferred_element_type=jnp.float32)
    pcnt = jnp.broadcast_to(jnp.sum(p, axis=1, keepdims=True), (N_GRAPHS, H))

    @pl.when(i == 0)
    def _():
        sums_ref[...] = psum
        cnts_ref[...] = pcnt

    @pl.when(i != 0)
    def _():
        sums_ref[...] += psum
        cnts_ref[...] += pcnt

    @pl.when(i == nb - 1)
    def _():
        pooled = sums_ref[...] / jnp.maximum(cnts_ref[...], 1.0)
        z = jnp.maximum(
            jnp.dot(pooled, w1_ref[...], preferred_element_type=jnp.float32) + b1_ref[...],
            0.0,
        )
        o_ref[...] = jnp.dot(z, w2_ref[...], preferred_element_type=jnp.float32) + b2_ref[...]


def _head(x, tab, batch3, w_en_x, b_en, w1, b1, w2, b2):
    return pl.pallas_call(
        _head_body,
        grid=(N // BN,),
        in_specs=[
            pl.BlockSpec((BN, D_NODE), lambda i: (i, 0)),
            pl.BlockSpec((BN, H), lambda i: (i, 0)),
            pl.BlockSpec((1, 1, BN), lambda i: (i, 0, 0)),
            pl.BlockSpec((D_NODE, H), lambda i: (0, 0)),
            pl.BlockSpec((1, H), lambda i: (0, 0)),
            pl.BlockSpec((H, H), lambda i: (0, 0)),
            pl.BlockSpec((1, H), lambda i: (0, 0)),
            pl.BlockSpec((H, 1), lambda i: (0, 0)),
            pl.BlockSpec((1, 1), lambda i: (0, 0)),
        ],
        out_specs=pl.BlockSpec((N_GRAPHS, 1), lambda i: (0, 0)),
        out_shape=jax.ShapeDtypeStruct((N_GRAPHS, 1), jnp.float32),
        scratch_shapes=[
            pltpu.VMEM((N_GRAPHS, H), jnp.float32),
            pltpu.VMEM((N_GRAPHS, H), jnp.float32),
        ],
    )(x, tab, batch3, w_en_x, b_en, w1, b1, w2, b2)


def _sc_scatter_body(hw_hbm, col_hbm, zeros_hbm, part_hbm, idx_v, rows_v, a_sh,
                     sem0, sem1):
    c = lax.axis_index("c")
    s = lax.axis_index("s")
    wid = s * SC_CORES + c
    tslice = pl.ds(s * NROWS_T, NROWS_T)
    base = wid * W_E
    sems = (sem0, sem1)

    def load(j, par):
        pltpu.async_copy(hw_hbm.at[pl.ds(base + j * CHUNK, CHUNK)],
                         rows_v.at[par], sems[par])

    def drain(par):
        pltpu.make_async_copy(hw_hbm.at[pl.ds(base, CHUNK)], rows_v.at[par],
                              sems[par]).wait()

    pltpu.async_copy(hw_hbm.at[pl.ds(base, CHUNK)], rows_v.at[0], sem0)
    pltpu.sync_copy(zeros_hbm.at[tslice], a_sh.at[tslice])
    pltpu.sync_copy(col_hbm.at[wid], idx_v)
    plsc.subcore_barrier()

    def pair(t, carry):
        j0 = 2 * t

        @pl.when(j0 + 1 < W_CH)
        def _():
            load(j0 + 1, 1)
        drain(0)
        pltpu.sync_copy(rows_v.at[0], a_sh.at[idx_v.at[j0]], add=True)

        @pl.when(j0 + 1 < W_CH)
        def _():
            @pl.when(j0 + 2 < W_CH)
            def _():
                load(j0 + 2, 0)
            drain(1)
            pltpu.sync_copy(rows_v.at[1], a_sh.at[idx_v.at[j0 + 1]], add=True)
        return carry

    lax.fori_loop(0, (W_CH + 1) // 2, pair, 0)
    plsc.subcore_barrier()
    pltpu.sync_copy(a_sh.at[tslice], part_hbm.at[c].at[tslice])


def _sc_scatter(hw, col3, zeros_n):
    mesh = plsc.VectorSubcoreMesh(core_axis_name="c", subcore_axis_name="s")
    f = pl.kernel(
        _sc_scatter_body,
        out_type=jax.ShapeDtypeStruct((SC_CORES, N_PAD, H), jnp.float32),
        mesh=mesh,
        scratch_types=[
            pltpu.VMEM((W_CH, CHUNK), jnp.int32),
            pltpu.VMEM((2, CHUNK, H), jnp.float32),
            pltpu.VMEM_SHARED((N_PAD, H), jnp.float32),
            pltpu.SemaphoreType.DMA,
            pltpu.SemaphoreType.DMA,
        ],
    )
    return f(hw, col3, zeros_n)


def _sc_gather_body(tab_hbm, row_hbm, out_hbm, idx_v, rows_v, a_sh, sem0, sem1):
    c = lax.axis_index("c")
    s = lax.axis_index("s")
    wid = s * SC_CORES + c
    tslice = pl.ds(s * NROWS_T, NROWS_T)
    pltpu.sync_copy(tab_hbm.at[tslice], a_sh.at[tslice])
    pltpu.sync_copy(row_hbm.at[wid], idx_v)
    plsc.subcore_barrier()
    base = wid * W_E
    sems = (sem0, sem1)

    def gath(j, par):
        pltpu.async_copy(a_sh.at[idx_v.at[j]], rows_v.at[par], sems[par])

    def drain(par):
        pltpu.make_async_copy(a_sh.at[idx_v.at[0]], rows_v.at[par],
                              sems[par]).wait()

    def wr(j, par):
        pltpu.sync_copy(rows_v.at[par],
                        out_hbm.at[pl.ds(base + j * CHUNK, CHUNK)])

    gath(0, 0)

    def pair(t, carry):
        j0 = 2 * t

        @pl.when(j0 + 1 < W_CH)
        def _():
            gath(j0 + 1, 1)
        drain(0)
        wr(j0, 0)

        @pl.when(j0 + 1 < W_CH)
        def _():
            @pl.when(j0 + 2 < W_CH)
            def _():
                gath(j0 + 2, 0)
            drain(1)
            wr(j0 + 1, 1)
        return carry

    lax.fori_loop(0, (W_CH + 1) // 2, pair, 0)


def _sc_gather(tab, row3):
    mesh = plsc.VectorSubcoreMesh(core_axis_name="c", subcore_axis_name="s")
    f = pl.kernel(
        _sc_gather_body,
        out_type=jax.ShapeDtypeStruct((ES, H), jnp.float32),
        mesh=mesh,
        scratch_types=[
            pltpu.VMEM((W_CH, CHUNK), jnp.int32),
            pltpu.VMEM((2, CHUNK, H), jnp.float32),
            pltpu.VMEM_SHARED((N_PAD, H), jnp.float32),
            pltpu.SemaphoreType.DMA,
            pltpu.SemaphoreType.DMA,
        ],
    )
    return f(tab, row3)


def kernel(x, edge_index, edge_attr, batch, W_ei, b_ei, W_conv, b_conv,
           W_en, b_en, W1, b1, W2, b2):
    row = edge_index[0]
    col = edge_index[1]
    def to_slot(a):
        return a.reshape(S, E2S, 2).transpose(0, 2, 1)
    row_t = to_slot(row)
    col_t = to_slot(col)
    ea_t = edge_attr.reshape(S, E2S, 2, D_EDGE).transpose(0, 2, 1, 3)
    ea_t = ea_t.astype(jnp.bfloat16)

    row3 = [row_t[s_].reshape(NW, W_CH, CHUNK) for s_ in range(S)]
    col3 = [col_t[s_].reshape(NW, W_CH, CHUNK) for s_ in range(S)]
    zeros_n = jnp.zeros((N_PAD, H), jnp.float32)
    batch3 = batch.reshape(N // BN, 1, BN)
    x_pad = jnp.pad(x, ((0, N_PAD - N), (0, 0))).astype(jnp.bfloat16)

    W_ei_x, W_ei_e = W_ei[:D_NODE], W_ei[D_NODE:]
    W_en_x, W_en_s = W_en[:D_NODE], W_en[D_NODE:]
    b_ei2 = b_ei.reshape(1, H)
    b_en2 = b_en.reshape(1, H)
    b1_2 = b1.reshape(1, H)
    b2_2 = b2.reshape(1, 1)

    xe = _node_mm(x_pad, W_ei_x)
    h0 = [None] * S
    hw = [None] * S
    for s_ in range(S):
        xg = _sc_gather(xe, row3[s_]).reshape(2, E2S, H)
        h0[s_], hw[s_] = _edge_init(xg, ea_t[s_], W_ei_e, b_ei2, W_conv[0])

    for l in range(DEPTH):
        parts = [_sc_scatter(hw[s_].reshape(ES, H), col3[s_], zeros_n)
                 for s_ in range(S)]
        tab = _combine4(parts[0], parts[1])
        w_next = W_conv[l + 1] if l + 1 < DEPTH else W_en_s
        for s_ in range(S):
            g = _sc_gather(tab, row3[s_]).reshape(2, E2S, H)
            hw[s_] = _layer(g, hw[s_], h0[s_], b_conv[l].reshape(1, H), w_next)

    parts = [_sc_scatter(hw[s_].reshape(ES, H), col3[s_], zeros_n)
             for s_ in range(S)]
    tab4 = _combine4(parts[0], parts[1])
    return _head(x, tab4[:N], batch3, W_en_x, b_en2, W1, b1_2, W2, b2_2)


# speedup vs baseline: 1.1637x; 1.1637x over previous
import jax
import jax.numpy as jnp
from jax import lax
from jax.experimental import pallas as pl
from jax.experimental.pallas import tpu as pltpu
from jax.experimental.pallas import tpu_sc as plsc

N = 10000
E = 320000
E2 = E // 2
D_NODE = 128
D_EDGE = 16
H = 128
DEPTH = 4
N_GRAPHS = 64

SC_CORES = 2
SC_TILES = 16
NW = SC_CORES * SC_TILES
CHUNK = 80
W_E = E // NW
W_CH = W_E // CHUNK
N_PAD = 10240
NROWS_T = N_PAD // SC_TILES
NCH_T = NROWS_T // CHUNK

BE = 4000
BN = 2000


def _node_mm_body(x_ref, w_ref, o_ref):
    o_ref[...] = jnp.dot(x_ref[...], w_ref[...].astype(jnp.bfloat16),
                         preferred_element_type=jnp.float32)


def _node_mm(x_pad, w):
    return pl.pallas_call(
        _node_mm_body,
        grid=(N_PAD // 2048,),
        in_specs=[
            pl.BlockSpec((2048, D_NODE), lambda i: (i, 0)),
            pl.BlockSpec((D_NODE, H), lambda i: (0, 0)),
        ],
        out_specs=pl.BlockSpec((2048, H), lambda i: (i, 0)),
        out_shape=jax.ShapeDtypeStruct((N_PAD, H), jnp.float32),
    )(x_pad, w)


def _edge_init_body(xg_ref, ea_ref, wee_ref, be_ref, wc0_ref, h0_ref, hw_ref):
    h0 = jnp.maximum(
        xg_ref[0].astype(jnp.float32)
        + jnp.dot(ea_ref[0], wee_ref[...].astype(jnp.bfloat16),
                  preferred_element_type=jnp.float32)
        + be_ref[...],
        0.0,
    ).astype(jnp.bfloat16)
    h0_ref[0] = h0
    hw_ref[0] = jnp.dot(h0, wc0_ref[...].astype(jnp.bfloat16),
                        preferred_element_type=jnp.float32)


def _edge_init(xg, ea, w_ei_e, b_ei, w_c0):
    return pl.pallas_call(
        _edge_init_body,
        grid=(2, E2 // BE),
        in_specs=[
            pl.BlockSpec((1, BE, H), lambda g, e: (g, e, 0)),
            pl.BlockSpec((1, BE, D_EDGE), lambda g, e: (g, e, 0)),
            pl.BlockSpec((D_EDGE, H), lambda g, e: (0, 0)),
            pl.BlockSpec((1, H), lambda g, e: (0, 0)),
            pl.BlockSpec((H, H), lambda g, e: (0, 0)),
        ],
        out_specs=[
            pl.BlockSpec((1, BE, H), lambda g, e: (g, e, 0)),
            pl.BlockSpec((1, BE, H), lambda g, e: (g, e, 0)),
        ],
        out_shape=[
            jax.ShapeDtypeStruct((2, E2, H), jnp.bfloat16),
            jax.ShapeDtypeStruct((2, E2, H), jnp.float32),
        ],
    )(xg, ea, w_ei_e, b_ei, w_c0)


def _layer_body(g_ref, hwr_ref, h0_ref, b_ref, w_ref, o_ref):
    h = jnp.maximum(
        g_ref[0].astype(jnp.float32) - hwr_ref[0].astype(jnp.float32)
        + b_ref[...] + h0_ref[0].astype(jnp.float32),
        0.0,
    ).astype(jnp.bfloat16)
    o_ref[0] = jnp.dot(h, w_ref[...].astype(jnp.bfloat16),
                       preferred_element_type=jnp.float32)


def _layer(g, hw, h0, b, w_next):
    return pl.pallas_call(
        _layer_body,
        grid=(2, E2 // BE),
        in_specs=[
            pl.BlockSpec((1, BE, H), lambda g_, e: (g_, e, 0)),
            pl.BlockSpec((1, BE, H), lambda g_, e: (1 - g_, e, 0)),
            pl.BlockSpec((1, BE, H), lambda g_, e: (g_, e, 0)),
            pl.BlockSpec((1, H), lambda g_, e: (0, 0)),
            pl.BlockSpec((H, H), lambda g_, e: (0, 0)),
        ],
        out_specs=pl.BlockSpec((1, BE, H), lambda g_, e: (g_, e, 0)),
        out_shape=jax.ShapeDtypeStruct((2, E2, H), jnp.float32),
    )(g, hw, h0, b, w_next)


def _head_body(x_ref, p0_ref, p1_ref, batch_ref, wen_ref, ben_ref, w1_ref,
               b1_ref, w2_ref, b2_ref, o_ref, sums_ref, cnts_ref):
    i = pl.program_id(0)
    nb = pl.num_programs(0)
    hn = jnp.maximum(
        jnp.dot(x_ref[...], wen_ref[...], preferred_element_type=jnp.float32)
        + p0_ref[0].astype(jnp.float32) + p1_ref[0].astype(jnp.float32)
        + ben_ref[...],
        0.0,
    )
    b = batch_ref[0, 0]
    p = (lax.broadcasted_iota(jnp.int32, (N_GRAPHS, BN), 0) == b[None, :]).astype(jnp.float32)
    psum = jnp.dot(p, hn, preferred_element_type=jnp.float32)
    pcnt = jnp.broadcast_to(jnp.sum(p, axis=1, keepdims=True), (N_GRAPHS, H))

    @pl.when(i == 0)
    def _():
        sums_ref[...] = psum
        cnts_ref[...] = pcnt

    @pl.when(i != 0)
    def _():
        sums_ref[...] += psum
        cnts_ref[...] += pcnt

    @pl.when(i == nb - 1)
    def _():
        pooled = sums_ref[...] / jnp.maximum(cnts_ref[...], 1.0)
        z = jnp.maximum(
            jnp.dot(pooled, w1_ref[...], preferred_element_type=jnp.float32) + b1_ref[...],
            0.0,
        )
        o_ref[...] = jnp.dot(z, w2_ref[...], preferred_element_type=jnp.float32) + b2_ref[...]


def _head(x, parts, batch3, w_en_x, b_en, w1, b1, w2, b2):
    return pl.pallas_call(
        _head_body,
        grid=(N // BN,),
        in_specs=[
            pl.BlockSpec((BN, D_NODE), lambda i: (i, 0)),
            pl.BlockSpec((1, BN, H), lambda i: (0, i, 0)),
            pl.BlockSpec((1, BN, H), lambda i: (1, i, 0)),
            pl.BlockSpec((1, 1, BN), lambda i: (i, 0, 0)),
            pl.BlockSpec((D_NODE, H), lambda i: (0, 0)),
            pl.BlockSpec((1, H), lambda i: (0, 0)),
            pl.BlockSpec((H, H), lambda i: (0, 0)),
            pl.BlockSpec((1, H), lambda i: (0, 0)),
            pl.BlockSpec((H, 1), lambda i: (0, 0)),
            pl.BlockSpec((1, 1), lambda i: (0, 0)),
        ],
        out_specs=pl.BlockSpec((N_GRAPHS, 1), lambda i: (0, 0)),
        out_shape=jax.ShapeDtypeStruct((N_GRAPHS, 1), jnp.float32),
        scratch_shapes=[
            pltpu.VMEM((N_GRAPHS, H), jnp.float32),
            pltpu.VMEM((N_GRAPHS, H), jnp.float32),
        ],
    )(x, parts, parts, batch3, w_en_x, b_en, w1, b1, w2, b2)


def _sc_scatgath_body(hw_hbm, col_hbm, row_hbm, zeros_hbm, iota_hbm,
                      g_hbm, part_hbm, idxv, itv, rows_v, a_sh,
                      sem0, sem1, semx):
    c = lax.axis_index("c")
    s = lax.axis_index("s")
    wid = s * SC_CORES + c
    tslice = pl.ds(s * NROWS_T, NROWS_T)
    base = wid * W_E
    sems = (sem0, sem1)

    def load(j, par):
        pltpu.async_copy(hw_hbm.at[pl.ds(base + j * CHUNK, CHUNK)],
                         rows_v.at[par], sems[par])

    def draini(par):
        pltpu.make_async_copy(hw_hbm.at[pl.ds(base, CHUNK)], rows_v.at[par],
                              sems[par]).wait()

    pltpu.async_copy(hw_hbm.at[pl.ds(base, CHUNK)], rows_v.at[0], sem0)
    pltpu.sync_copy(zeros_hbm.at[tslice], a_sh.at[tslice])
    pltpu.sync_copy(col_hbm.at[wid], idxv)
    pltpu.sync_copy(iota_hbm.at[s], itv)
    plsc.subcore_barrier()

    def spair(t, carry):
        j0 = 2 * t

        @pl.when(j0 + 1 < W_CH)
        def _():
            load(j0 + 1, 1)
        draini(0)
        pltpu.sync_copy(rows_v.at[0], a_sh.at[idxv.at[j0]], add=True)

        @pl.when(j0 + 1 < W_CH)
        def _():
            @pl.when(j0 + 2 < W_CH)
            def _():
                load(j0 + 2, 0)
            draini(1)
            pltpu.sync_copy(rows_v.at[1], a_sh.at[idxv.at[j0 + 1]], add=True)
        return carry

    lax.fori_loop(0, (W_CH + 1) // 2, spair, 0)
    plsc.subcore_barrier()
    pltpu.sync_copy(a_sh.at[tslice], part_hbm.at[c].at[tslice])
    pltpu.sync_copy(row_hbm.at[wid], idxv)
    pltpu.semaphore_signal(semx, 1, core_index=1 - c)
    pltpu.semaphore_wait(semx, 1)
    rbase = s * NROWS_T
    for k in range(NCH_T):
        pltpu.sync_copy(part_hbm.at[1 - c].at[pl.ds(rbase + k * CHUNK, CHUNK)],
                        rows_v.at[k % 2])
        pltpu.sync_copy(rows_v.at[k % 2], a_sh.at[itv.at[k]], add=True)
    plsc.subcore_barrier()

    def gath(j, par):
        pltpu.async_copy(a_sh.at[idxv.at[j]], rows_v.at[par], sems[par])

    def draing(par):
        pltpu.make_async_copy(a_sh.at[idxv.at[0]], rows_v.at[par],
                              sems[par]).wait()

    def wr(j, par):
        pltpu.sync_copy(rows_v.at[par],
                        g_hbm.at[pl.ds(base + j * CHUNK, CHUNK)])

    gath(0, 0)

    def gpair(t, carry):
        j0 = 2 * t

        @pl.when(j0 + 1 < W_CH)
        def _():
            gath(j0 + 1, 1)
        draing(0)
        wr(j0, 0)

        @pl.when(j0 + 1 < W_CH)
        def _():
            @pl.when(j0 + 2 < W_CH)
            def _():
                gath(j0 + 2, 0)
            draing(1)
            wr(j0 + 1, 1)
        return carry

    lax.fori_loop(0, (W_CH + 1) // 2, gpair, 0)


def _sc_scatgath(hw, col3, row3, zeros_n, iota_n):
    mesh = plsc.VectorSubcoreMesh(core_axis_name="c", subcore_axis_name="s")
    f = pl.kernel(
        _sc_scatgath_body,
        out_type=[
            jax.ShapeDtypeStruct((E, H), jnp.float32),
            jax.ShapeDtypeStruct((SC_CORES, N_PAD, H), jnp.float32),
        ],
        mesh=mesh,
        scratch_types=[
            pltpu.VMEM((W_CH, CHUNK), jnp.int32),
            pltpu.VMEM((NCH_T, CHUNK), jnp.int32),
            pltpu.VMEM((2, CHUNK, H), jnp.float32),
            pltpu.VMEM_SHARED((N_PAD, H), jnp.float32),
            pltpu.SemaphoreType.DMA,
            pltpu.SemaphoreType.DMA,
            pltpu.SemaphoreType.REGULAR,
        ],
    )
    return f(hw, col3, row3, zeros_n, iota_n)


def _sc_scatter_body(hw_hbm, col_hbm, zeros_hbm, part_hbm, idx_v, rows_v, a_sh,
                     sem0, sem1):
    c = lax.axis_index("c")
    s = lax.axis_index("s")
    wid = s * SC_CORES + c
    tslice = pl.ds(s * NROWS_T, NROWS_T)
    base = wid * W_E
    sems = (sem0, sem1)

    def load(j, par):
        pltpu.async_copy(hw_hbm.at[pl.ds(base + j * CHUNK, CHUNK)],
                         rows_v.at[par], sems[par])

    def drain(par):
        pltpu.make_async_copy(hw_hbm.at[pl.ds(base, CHUNK)], rows_v.at[par],
                              sems[par]).wait()

    pltpu.async_copy(hw_hbm.at[pl.ds(base, CHUNK)], rows_v.at[0], sem0)
    pltpu.sync_copy(zeros_hbm.at[tslice], a_sh.at[tslice])
    pltpu.sync_copy(col_hbm.at[wid], idx_v)
    plsc.subcore_barrier()

    def pair(t, carry):
        j0 = 2 * t

        @pl.when(j0 + 1 < W_CH)
        def _():
            load(j0 + 1, 1)
        drain(0)
        pltpu.sync_copy(rows_v.at[0], a_sh.at[idx_v.at[j0]], add=True)

        @pl.when(j0 + 1 < W_CH)
        def _():
            @pl.when(j0 + 2 < W_CH)
            def _():
                load(j0 + 2, 0)
            drain(1)
            pltpu.sync_copy(rows_v.at[1], a_sh.at[idx_v.at[j0 + 1]], add=True)
        return carry

    lax.fori_loop(0, (W_CH + 1) // 2, pair, 0)
    plsc.subcore_barrier()
    pltpu.sync_copy(a_sh.at[tslice], part_hbm.at[c].at[tslice])


def _sc_scatter(hw, col3, zeros_n):
    mesh = plsc.VectorSubcoreMesh(core_axis_name="c", subcore_axis_name="s")
    f = pl.kernel(
        _sc_scatter_body,
        out_type=jax.ShapeDtypeStruct((SC_CORES, N_PAD, H), jnp.float32),
        mesh=mesh,
        scratch_types=[
            pltpu.VMEM((W_CH, CHUNK), jnp.int32),
            pltpu.VMEM((2, CHUNK, H), jnp.float32),
            pltpu.VMEM_SHARED((N_PAD, H), jnp.float32),
            pltpu.SemaphoreType.DMA,
            pltpu.SemaphoreType.DMA,
        ],
    )
    return f(hw, col3, zeros_n)


def _sc_gather_body(tab_hbm, row_hbm, out_hbm, idx_v, rows_v, a_sh, sem0, sem1):
    c = lax.axis_index("c")
    s = lax.axis_index("s")
    wid = s * SC_CORES + c
    tslice = pl.ds(s * NROWS_T, NROWS_T)
    pltpu.sync_copy(tab_hbm.at[tslice], a_sh.at[tslice])
    pltpu.sync_copy(row_hbm.at[wid], idx_v)
    plsc.subcore_barrier()
    base = wid * W_E
    sems = (sem0, sem1)

    def gath(j, par):
        pltpu.async_copy(a_sh.at[idx_v.at[j]], rows_v.at[par], sems[par])

    def drain(par):
        pltpu.make_async_copy(a_sh.at[idx_v.at[0]], rows_v.at[par],
                              sems[par]).wait()

    def wr(j, par):
        pltpu.sync_copy(rows_v.at[par],
                        out_hbm.at[pl.ds(base + j * CHUNK, CHUNK)])

    gath(0, 0)

    def pair(t, carry):
        j0 = 2 * t

        @pl.when(j0 + 1 < W_CH)
        def _():
            gath(j0 + 1, 1)
        drain(0)
        wr(j0, 0)

        @pl.when(j0 + 1 < W_CH)
        def _():
            @pl.when(j0 + 2 < W_CH)
            def _():
                gath(j0 + 2, 0)
            drain(1)
            wr(j0 + 1, 1)
        return carry

    lax.fori_loop(0, (W_CH + 1) // 2, pair, 0)


def _sc_gather(tab_i32, row3):
    mesh = plsc.VectorSubcoreMesh(core_axis_name="c", subcore_axis_name="s")
    f = pl.kernel(
        _sc_gather_body,
        out_type=jax.ShapeDtypeStruct((E, H), jnp.float32),
        mesh=mesh,
        scratch_types=[
            pltpu.VMEM((W_CH, CHUNK), jnp.int32),
            pltpu.VMEM((2, CHUNK, H), jnp.float32),
            pltpu.VMEM_SHARED((N_PAD, H), jnp.float32),
            pltpu.SemaphoreType.DMA,
            pltpu.SemaphoreType.DMA,
        ],
    )
    return f(tab_i32, row3)


def _combine_body(p0_ref, p1_ref, o_ref):
    o_ref[...] = p0_ref[0] + p1_ref[0]


def _combine(parts):
    return pl.pallas_call(
        _combine_body,
        grid=(N_PAD // 2048,),
        in_specs=[
            pl.BlockSpec((1, 2048, H), lambda i: (0, i, 0)),
            pl.BlockSpec((1, 2048, H), lambda i: (1, i, 0)),
        ],
        out_specs=pl.BlockSpec((2048, H), lambda i: (i, 0)),
        out_shape=jax.ShapeDtypeStruct((N_PAD, H), jnp.float32),
    )(parts, parts)


def _bf_to_i32(a):
    return lax.bitcast_convert_type(
        a.reshape(a.shape[:-1] + (H // 2, 2)), jnp.int32)


def _i32_to_bf(a):
    return lax.bitcast_convert_type(a, jnp.bfloat16).reshape(2, E2, H)


def kernel(x, edge_index, edge_attr, batch, W_ei, b_ei, W_conv, b_conv,
           W_en, b_en, W1, b1, W2, b2):
    row = edge_index[0]
    col = edge_index[1]
    row_s = row.reshape(E2, 2).swapaxes(0, 1).reshape(E)
    col_s = col.reshape(E2, 2).swapaxes(0, 1).reshape(E)
    ea_s = edge_attr.reshape(E2, 2, D_EDGE).swapaxes(0, 1).astype(jnp.bfloat16)

    row3 = row_s.reshape(NW, W_CH, CHUNK)
    col3 = col_s.reshape(NW, W_CH, CHUNK)
    iota_n = jnp.arange(N_PAD, dtype=jnp.int32).reshape(SC_TILES, NCH_T, CHUNK)
    zeros_n = jnp.zeros((N_PAD, H), jnp.float32)
    batch3 = batch.reshape(N // BN, 1, BN)
    x_pad = jnp.pad(x, ((0, N_PAD - N), (0, 0))).astype(jnp.bfloat16)

    W_ei_x, W_ei_e = W_ei[:D_NODE], W_ei[D_NODE:]
    W_en_x, W_en_s = W_en[:D_NODE], W_en[D_NODE:]
    b_ei2 = b_ei.reshape(1, H)
    b_en2 = b_en.reshape(1, H)
    b1_2 = b1.reshape(1, H)
    b2_2 = b2.reshape(1, 1)

    xe = _node_mm(x_pad, W_ei_x)
    xg = _sc_gather(xe, row3).reshape(2, E2, H)
    h0, hw = _edge_init(xg, ea_s, W_ei_e, b_ei2, W_conv[0])

    for l in range(DEPTH):
        g, _unused = _sc_scatgath(hw.reshape(E, H), col3, row3, zeros_n, iota_n)
        g = g.reshape(2, E2, H)
        w_next = W_conv[l + 1] if l + 1 < DEPTH else W_en_s
        hw = _layer(g, hw, h0, b_conv[l].reshape(1, H), w_next)

    parts4 = _sc_scatter(hw.reshape(E, H), col3, zeros_n)
    return _head(x, parts4, batch3, W_en_x, b_en2, W1, b1_2, W2, b2_2)


# speedup vs baseline: 1.1714x; 1.0066x over previous
import jax
import jax.numpy as jnp
from jax import lax
from jax.experimental import pallas as pl
from jax.experimental.pallas import tpu as pltpu
from jax.experimental.pallas import tpu_sc as plsc

N = 10000
E = 320000
E2 = E // 2
D_NODE = 128
D_EDGE = 16
H = 128
DEPTH = 4
N_GRAPHS = 64

SC_CORES = 2
SC_TILES = 16
NW = SC_CORES * SC_TILES
CHUNK = 80
W_E = E // NW
W_CH = W_E // CHUNK
N_PAD = 10240
NROWS_T = N_PAD // SC_TILES
NCH_T = NROWS_T // CHUNK

BE = 8000
BN = 2000


def _node_mm_body(x_ref, w_ref, o_ref):
    o_ref[...] = jnp.dot(x_ref[...], w_ref[...].astype(jnp.bfloat16),
                         preferred_element_type=jnp.float32)


def _node_mm(x_pad, w):
    return pl.pallas_call(
        _node_mm_body,
        grid=(N_PAD // 2048,),
        in_specs=[
            pl.BlockSpec((2048, D_NODE), lambda i: (i, 0)),
            pl.BlockSpec((D_NODE, H), lambda i: (0, 0)),
        ],
        out_specs=pl.BlockSpec((2048, H), lambda i: (i, 0)),
        out_shape=jax.ShapeDtypeStruct((N_PAD, H), jnp.float32),
    )(x_pad, w)


def _edge_init_body(xg_ref, ea_ref, wee_ref, be_ref, wc0_ref, h0_ref, hw_ref):
    h0 = jnp.maximum(
        xg_ref[0].astype(jnp.float32)
        + jnp.dot(ea_ref[0], wee_ref[...].astype(jnp.bfloat16),
                  preferred_element_type=jnp.float32)
        + be_ref[...],
        0.0,
    ).astype(jnp.bfloat16)
    h0_ref[0] = h0
    hw_ref[0] = jnp.dot(h0, wc0_ref[...].astype(jnp.bfloat16),
                        preferred_element_type=jnp.float32)


def _edge_init(xg, ea, w_ei_e, b_ei, w_c0):
    return pl.pallas_call(
        _edge_init_body,
        grid=(2, E2 // BE),
        in_specs=[
            pl.BlockSpec((1, BE, H), lambda g, e: (g, e, 0)),
            pl.BlockSpec((1, BE, D_EDGE), lambda g, e: (g, e, 0)),
            pl.BlockSpec((D_EDGE, H), lambda g, e: (0, 0)),
            pl.BlockSpec((1, H), lambda g, e: (0, 0)),
            pl.BlockSpec((H, H), lambda g, e: (0, 0)),
        ],
        out_specs=[
            pl.BlockSpec((1, BE, H), lambda g, e: (g, e, 0)),
            pl.BlockSpec((1, BE, H), lambda g, e: (g, e, 0)),
        ],
        out_shape=[
            jax.ShapeDtypeStruct((2, E2, H), jnp.bfloat16),
            jax.ShapeDtypeStruct((2, E2, H), jnp.float32),
        ],
    )(xg, ea, w_ei_e, b_ei, w_c0)


def _layer_body(g_ref, hwr_ref, h0_ref, b_ref, w_ref, o_ref):
    h = jnp.maximum(
        g_ref[0].astype(jnp.float32) - hwr_ref[0].astype(jnp.float32)
        + b_ref[...] + h0_ref[0].astype(jnp.float32),
        0.0,
    ).astype(jnp.bfloat16)
    o_ref[0] = jnp.dot(h, w_ref[...].astype(jnp.bfloat16),
                       preferred_element_type=jnp.float32)


def _layer(g, hw, h0, b, w_next):
    return pl.pallas_call(
        _layer_body,
        grid=(2, E2 // BE),
        in_specs=[
            pl.BlockSpec((1, BE, H), lambda g_, e: (g_, e, 0)),
            pl.BlockSpec((1, BE, H), lambda g_, e: (1 - g_, e, 0)),
            pl.BlockSpec((1, BE, H), lambda g_, e: (g_, e, 0)),
            pl.BlockSpec((1, H), lambda g_, e: (0, 0)),
            pl.BlockSpec((H, H), lambda g_, e: (0, 0)),
        ],
        out_specs=pl.BlockSpec((1, BE, H), lambda g_, e: (g_, e, 0)),
        out_shape=jax.ShapeDtypeStruct((2, E2, H), jnp.float32),
    )(g, hw, h0, b, w_next)


def _head_body(x_ref, p0_ref, p1_ref, batch_ref, wen_ref, ben_ref, w1_ref,
               b1_ref, w2_ref, b2_ref, o_ref, sums_ref, cnts_ref):
    i = pl.program_id(0)
    nb = pl.num_programs(0)
    hn = jnp.maximum(
        jnp.dot(x_ref[...], wen_ref[...], preferred_element_type=jnp.float32)
        + p0_ref[0].astype(jnp.float32) + p1_ref[0].astype(jnp.float32)
        + ben_ref[...],
        0.0,
    )
    b = batch_ref[0, 0]
    p = (lax.broadcasted_iota(jnp.int32, (N_GRAPHS, BN), 0) == b[None, :]).astype(jnp.float32)
    psum = jnp.dot(p, hn, preferred_element_type=jnp.float32)
    pcnt = jnp.broadcast_to(jnp.sum(p, axis=1, keepdims=True), (N_GRAPHS, H))

    @pl.when(i == 0)
    def _():
        sums_ref[...] = psum
        cnts_ref[...] = pcnt

    @pl.when(i != 0)
    def _():
        sums_ref[...] += psum
        cnts_ref[...] += pcnt

    @pl.when(i == nb - 1)
    def _():
        pooled = sums_ref[...] / jnp.maximum(cnts_ref[...], 1.0)
        z = jnp.maximum(
            jnp.dot(pooled, w1_ref[...], preferred_element_type=jnp.float32) + b1_ref[...],
            0.0,
        )
        o_ref[...] = jnp.dot(z, w2_ref[...], preferred_element_type=jnp.float32) + b2_ref[...]


def _head(x, parts, batch3, w_en_x, b_en, w1, b1, w2, b2):
    return pl.pallas_call(
        _head_body,
        grid=(N // BN,),
        in_specs=[
            pl.BlockSpec((BN, D_NODE), lambda i: (i, 0)),
            pl.BlockSpec((1, BN, H), lambda i: (0, i, 0)),
            pl.BlockSpec((1, BN, H), lambda i: (1, i, 0)),
            pl.BlockSpec((1, 1, BN), lambda i: (i, 0, 0)),
            pl.BlockSpec((D_NODE, H), lambda i: (0, 0)),
            pl.BlockSpec((1, H), lambda i: (0, 0)),
            pl.BlockSpec((H, H), lambda i: (0, 0)),
            pl.BlockSpec((1, H), lambda i: (0, 0)),
            pl.BlockSpec((H, 1), lambda i: (0, 0)),
            pl.BlockSpec((1, 1), lambda i: (0, 0)),
        ],
        out_specs=pl.BlockSpec((N_GRAPHS, 1), lambda i: (0, 0)),
        out_shape=jax.ShapeDtypeStruct((N_GRAPHS, 1), jnp.float32),
        scratch_shapes=[
            pltpu.VMEM((N_GRAPHS, H), jnp.float32),
            pltpu.VMEM((N_GRAPHS, H), jnp.float32),
        ],
    )(x, parts, parts, batch3, w_en_x, b_en, w1, b1, w2, b2)


def _sc_scatgath_body(hw_hbm, col_hbm, row_hbm, zeros_hbm, iota_hbm,
                      g_hbm, part_hbm, idxv, itv, rows_v, a_sh,
                      sem0, sem1, semx):
    c = lax.axis_index("c")
    s = lax.axis_index("s")
    wid = s * SC_CORES + c
    tslice = pl.ds(s * NROWS_T, NROWS_T)
    base = wid * W_E
    sems = (sem0, sem1)

    def load(j, par):
        pltpu.async_copy(hw_hbm.at[pl.ds(base + j * CHUNK, CHUNK)],
                         rows_v.at[par], sems[par])

    def draini(par):
        pltpu.make_async_copy(hw_hbm.at[pl.ds(base, CHUNK)], rows_v.at[par],
                              sems[par]).wait()

    pltpu.async_copy(hw_hbm.at[pl.ds(base, CHUNK)], rows_v.at[0], sem0)
    pltpu.sync_copy(zeros_hbm.at[tslice], a_sh.at[tslice])
    pltpu.sync_copy(col_hbm.at[wid], idxv)
    pltpu.sync_copy(iota_hbm.at[s], itv)
    plsc.subcore_barrier()

    def spair(t, carry):
        j0 = 2 * t

        @pl.when(j0 + 1 < W_CH)
        def _():
            load(j0 + 1, 1)
        draini(0)
        pltpu.sync_copy(rows_v.at[0], a_sh.at[idxv.at[j0]], add=True)

        @pl.when(j0 + 1 < W_CH)
        def _():
            @pl.when(j0 + 2 < W_CH)
            def _():
                load(j0 + 2, 0)
            draini(1)
            pltpu.sync_copy(rows_v.at[1], a_sh.at[idxv.at[j0 + 1]], add=True)
        return carry

    lax.fori_loop(0, (W_CH + 1) // 2, spair, 0)
    plsc.subcore_barrier()
    pltpu.sync_copy(a_sh.at[tslice], part_hbm.at[c].at[tslice])
    pltpu.sync_copy(row_hbm.at[wid], idxv)
    pltpu.semaphore_signal(semx, 1, core_index=1 - c)
    pltpu.semaphore_wait(semx, 1)
    rbase = s * NROWS_T
    for k in range(NCH_T):
        pltpu.sync_copy(part_hbm.at[1 - c].at[pl.ds(rbase + k * CHUNK, CHUNK)],
                        rows_v.at[k % 2])
        pltpu.sync_copy(rows_v.at[k % 2], a_sh.at[itv.at[k]], add=True)
    plsc.subcore_barrier()

    def gath(j, par):
        pltpu.async_copy(a_sh.at[idxv.at[j]], rows_v.at[par], sems[par])

    def draing(par):
        pltpu.make_async_copy(a_sh.at[idxv.at[0]], rows_v.at[par],
                              sems[par]).wait()

    def wr(j, par):
        pltpu.sync_copy(rows_v.at[par],
                        g_hbm.at[pl.ds(base + j * CHUNK, CHUNK)])

    gath(0, 0)

    def gpair(t, carry):
        j0 = 2 * t

        @pl.when(j0 + 1 < W_CH)
        def _():
            gath(j0 + 1, 1)
        draing(0)
        wr(j0, 0)

        @pl.when(j0 + 1 < W_CH)
        def _():
            @pl.when(j0 + 2 < W_CH)
            def _():
                gath(j0 + 2, 0)
            draing(1)
            wr(j0 + 1, 1)
        return carry

    lax.fori_loop(0, (W_CH + 1) // 2, gpair, 0)


def _sc_scatgath(hw, col3, row3, zeros_n, iota_n):
    mesh = plsc.VectorSubcoreMesh(core_axis_name="c", subcore_axis_name="s")
    f = pl.kernel(
        _sc_scatgath_body,
        out_type=[
            jax.ShapeDtypeStruct((E, H), jnp.float32),
            jax.ShapeDtypeStruct((SC_CORES, N_PAD, H), jnp.float32),
        ],
        mesh=mesh,
        scratch_types=[
            pltpu.VMEM((W_CH, CHUNK), jnp.int32),
            pltpu.VMEM((NCH_T, CHUNK), jnp.int32),
            pltpu.VMEM((2, CHUNK, H), jnp.float32),
            pltpu.VMEM_SHARED((N_PAD, H), jnp.float32),
            pltpu.SemaphoreType.DMA,
            pltpu.SemaphoreType.DMA,
            pltpu.SemaphoreType.REGULAR,
        ],
    )
    return f(hw, col3, row3, zeros_n, iota_n)


def _sc_scatter_body(hw_hbm, col_hbm, zeros_hbm, part_hbm, idx_v, rows_v, a_sh,
                     sem0, sem1):
    c = lax.axis_index("c")
    s = lax.axis_index("s")
    wid = s * SC_CORES + c
    tslice = pl.ds(s * NROWS_T, NROWS_T)
    base = wid * W_E
    sems = (sem0, sem1)

    def load(j, par):
        pltpu.async_copy(hw_hbm.at[pl.ds(base + j * CHUNK, CHUNK)],
                         rows_v.at[par], sems[par])

    def drain(par):
        pltpu.make_async_copy(hw_hbm.at[pl.ds(base, CHUNK)], rows_v.at[par],
                              sems[par]).wait()

    pltpu.async_copy(hw_hbm.at[pl.ds(base, CHUNK)], rows_v.at[0], sem0)
    pltpu.sync_copy(zeros_hbm.at[tslice], a_sh.at[tslice])
    pltpu.sync_copy(col_hbm.at[wid], idx_v)
    plsc.subcore_barrier()

    def pair(t, carry):
        j0 = 2 * t

        @pl.when(j0 + 1 < W_CH)
        def _():
            load(j0 + 1, 1)
        drain(0)
        pltpu.sync_copy(rows_v.at[0], a_sh.at[idx_v.at[j0]], add=True)

        @pl.when(j0 + 1 < W_CH)
        def _():
            @pl.when(j0 + 2 < W_CH)
            def _():
                load(j0 + 2, 0)
            drain(1)
            pltpu.sync_copy(rows_v.at[1], a_sh.at[idx_v.at[j0 + 1]], add=True)
        return carry

    lax.fori_loop(0, (W_CH + 1) // 2, pair, 0)
    plsc.subcore_barrier()
    pltpu.sync_copy(a_sh.at[tslice], part_hbm.at[c].at[tslice])


def _sc_scatter(hw, col3, zeros_n):
    mesh = plsc.VectorSubcoreMesh(core_axis_name="c", subcore_axis_name="s")
    f = pl.kernel(
        _sc_scatter_body,
        out_type=jax.ShapeDtypeStruct((SC_CORES, N_PAD, H), jnp.float32),
        mesh=mesh,
        scratch_types=[
            pltpu.VMEM((W_CH, CHUNK), jnp.int32),
            pltpu.VMEM((2, CHUNK, H), jnp.float32),
            pltpu.VMEM_SHARED((N_PAD, H), jnp.float32),
            pltpu.SemaphoreType.DMA,
            pltpu.SemaphoreType.DMA,
        ],
    )
    return f(hw, col3, zeros_n)


def _sc_gather_body(tab_hbm, row_hbm, out_hbm, idx_v, rows_v, a_sh, sem0, sem1):
    c = lax.axis_index("c")
    s = lax.axis_index("s")
    wid = s * SC_CORES + c
    tslice = pl.ds(s * NROWS_T, NROWS_T)
    pltpu.sync_copy(tab_hbm.at[tslice], a_sh.at[tslice])
    pltpu.sync_copy(row_hbm.at[wid], idx_v)
    plsc.subcore_barrier()
    base = wid * W_E
    sems = (sem0, sem1)

    def gath(j, par):
        pltpu.async_copy(a_sh.at[idx_v.at[j]], rows_v.at[par], sems[par])

    def drain(par):
        pltpu.make_async_copy(a_sh.at[idx_v.at[0]], rows_v.at[par],
                              sems[par]).wait()

    def wr(j, par):
        pltpu.sync_copy(rows_v.at[par],
                        out_hbm.at[pl.ds(base + j * CHUNK, CHUNK)])

    gath(0, 0)

    def pair(t, carry):
        j0 = 2 * t

        @pl.when(j0 + 1 < W_CH)
        def _():
            gath(j0 + 1, 1)
        drain(0)
        wr(j0, 0)

        @pl.when(j0 + 1 < W_CH)
        def _():
            @pl.when(j0 + 2 < W_CH)
            def _():
                gath(j0 + 2, 0)
            drain(1)
            wr(j0 + 1, 1)
        return carry

    lax.fori_loop(0, (W_CH + 1) // 2, pair, 0)


def _sc_gather(tab_i32, row3):
    mesh = plsc.VectorSubcoreMesh(core_axis_name="c", subcore_axis_name="s")
    f = pl.kernel(
        _sc_gather_body,
        out_type=jax.ShapeDtypeStruct((E, H), jnp.float32),
        mesh=mesh,
        scratch_types=[
            pltpu.VMEM((W_CH, CHUNK), jnp.int32),
            pltpu.VMEM((2, CHUNK, H), jnp.float32),
            pltpu.VMEM_SHARED((N_PAD, H), jnp.float32),
            pltpu.SemaphoreType.DMA,
            pltpu.SemaphoreType.DMA,
        ],
    )
    return f(tab_i32, row3)


def _combine_body(p0_ref, p1_ref, o_ref):
    o_ref[...] = p0_ref[0] + p1_ref[0]


def _combine(parts):
    return pl.pallas_call(
        _combine_body,
        grid=(N_PAD // 2048,),
        in_specs=[
            pl.BlockSpec((1, 2048, H), lambda i: (0, i, 0)),
            pl.BlockSpec((1, 2048, H), lambda i: (1, i, 0)),
        ],
        out_specs=pl.BlockSpec((2048, H), lambda i: (i, 0)),
        out_shape=jax.ShapeDtypeStruct((N_PAD, H), jnp.float32),
    )(parts, parts)


def _bf_to_i32(a):
    return lax.bitcast_convert_type(
        a.reshape(a.shape[:-1] + (H // 2, 2)), jnp.int32)


def _i32_to_bf(a):
    return lax.bitcast_convert_type(a, jnp.bfloat16).reshape(2, E2, H)


def kernel(x, edge_index, edge_attr, batch, W_ei, b_ei, W_conv, b_conv,
           W_en, b_en, W1, b1, W2, b2):
    row = edge_index[0]
    col = edge_index[1]
    row_s = row.reshape(E2, 2).swapaxes(0, 1).reshape(E)
    col_s = col.reshape(E2, 2).swapaxes(0, 1).reshape(E)
    ea_s = edge_attr.reshape(E2, 2, D_EDGE).swapaxes(0, 1).astype(jnp.bfloat16)

    row3 = row_s.reshape(NW, W_CH, CHUNK)
    col3 = col_s.reshape(NW, W_CH, CHUNK)
    iota_n = jnp.arange(N_PAD, dtype=jnp.int32).reshape(SC_TILES, NCH_T, CHUNK)
    zeros_n = jnp.zeros((N_PAD, H), jnp.float32)
    batch3 = batch.reshape(N // BN, 1, BN)
    x_pad = jnp.pad(x, ((0, N_PAD - N), (0, 0))).astype(jnp.bfloat16)

    W_ei_x, W_ei_e = W_ei[:D_NODE], W_ei[D_NODE:]
    W_en_x, W_en_s = W_en[:D_NODE], W_en[D_NODE:]
    b_ei2 = b_ei.reshape(1, H)
    b_en2 = b_en.reshape(1, H)
    b1_2 = b1.reshape(1, H)
    b2_2 = b2.reshape(1, 1)

    xe = _node_mm(x_pad, W_ei_x)
    xg = _sc_gather(xe, row3).reshape(2, E2, H)
    h0, hw = _edge_init(xg, ea_s, W_ei_e, b_ei2, W_conv[0])

    for l in range(DEPTH):
        g, _unused = _sc_scatgath(hw.reshape(E, H), col3, row3, zeros_n, iota_n)
        g = g.reshape(2, E2, H)
        w_next = W_conv[l + 1] if l + 1 < DEPTH else W_en_s
        hw = _layer(g, hw, h0, b_conv[l].reshape(1, H), w_next)

    parts4 = _sc_scatter(hw.reshape(E, H), col3, zeros_n)
    return _head(x, parts4, batch3, W_en_x, b_en2, W1, b1_2, W2, b2_2)
